# baseline (speedup 1.0000x reference)
import functools

import jax
import jax.numpy as jnp
from jax import lax
from jax.experimental import pallas as pl
from jax.experimental.pallas import tpu as pltpu
from jax.experimental.pallas import tpu_sc as plsc

F32 = jnp.float32


def _unpack_bf16_pair(w):
    m = w.shape[1] // 2
    u = jax.lax.bitcast_convert_type(w, jnp.uint32)
    lo = jax.lax.bitcast_convert_type(u << 16, F32)
    hi = jax.lax.bitcast_convert_type(u & jnp.uint32(0xFFFF0000), F32)
    return jnp.concatenate(
        [jnp.concatenate([lo[:, :m], hi[:, :m]], axis=1),
         jnp.concatenate([lo[:, m:], hi[:, m:]], axis=1)], axis=0)


def _precompute_body(h, w1a, w1b, be1, outa, outb):
    hv = h[...]
    outa[...] = jnp.dot(hv, w1a[...], preferred_element_type=F32) + be1[...]
    outb[...] = jnp.dot(hv, w1b[...], preferred_element_type=F32)


def _edge_body(ga, gb, r2, ea, w1e, w1r, w2, b2, wat, ba, out):
    radial = jnp.transpose(jnp.sqrt(r2[...])[0])
    z = (_unpack_bf16_pair(ga[...]) + _unpack_bf16_pair(gb[...])
         + radial * w1r[...]
         + jnp.dot(ea[...], w1e[...], preferred_element_type=F32))
    m = z * jax.nn.sigmoid(z)
    y = jnp.dot(m, w2[...], preferred_element_type=F32) + b2[...]
    m2 = y * jax.nn.sigmoid(y)
    att_logit = jnp.sum(m2 * wat[...], axis=1, keepdims=True) + ba[...]
    out[...] = m2 * jax.nn.sigmoid(att_logit)


def _node_body(h, s0, s1, wh1a, wh1b, bh1, wh2, bh2, out):
    hv = h[...]
    s = s0[...] + s1[...]
    t = (jnp.dot(hv, wh1a[...], preferred_element_type=F32)
         + jnp.dot(s, wh1b[...], preferred_element_type=F32) + bh1[...])
    t = t * jax.nn.sigmoid(t)
    out[...] = hv + jnp.dot(t, wh2[...], preferred_element_type=F32) + bh2[...]


def _rne16(v):
    b = lax.bitcast_convert_type(v, jnp.uint32)
    return (b + jnp.uint32(0x7FFF) + ((b >> 16) & jnp.uint32(1))) >> 16


def _make_gather(n, e, d):
    info = plsc.get_sparse_core_info()
    nc, ns, nl = info.num_cores, info.num_subcores, info.num_lanes
    nw = nc * ns
    epw = e // nw
    chunk = 80
    nchunk = epw // chunk
    npairs = (nchunk - 1) // 2
    groups = chunk // nl
    mesh = plsc.VectorSubcoreMesh(core_axis_name="c", subcore_axis_name="s")

    @functools.partial(
        pl.kernel, mesh=mesh,
        out_type=[jax.ShapeDtypeStruct((e // 2, d), jnp.int32),
                  jax.ShapeDtypeStruct((e // 2, d), jnp.int32),
                  jax.ShapeDtypeStruct((e,), F32)],
        scratch_types=[pltpu.VMEM((chunk,), jnp.int32),
                       pltpu.VMEM((chunk,), jnp.int32),
                       pltpu.VMEM((chunk,), jnp.int32),
                       pltpu.VMEM((chunk,), jnp.int32),
                       pltpu.VMEM((chunk, d), F32),
                       pltpu.VMEM((chunk, d), F32),
                       pltpu.VMEM((chunk, d), F32),
                       pltpu.VMEM((chunk, d), F32),
                       pltpu.VMEM((chunk // 2, d), jnp.int32),
                       pltpu.VMEM((chunk // 2, d), jnp.int32),
                       pltpu.VMEM((chunk // 2, d), jnp.int32),
                       pltpu.VMEM((chunk // 2, d), jnp.int32),
                       pltpu.VMEM((chunk,), F32),
                       pltpu.VMEM((chunk,), F32),
                       pltpu.VMEM((n,), F32),
                       pltpu.VMEM((n,), F32),
                       pltpu.VMEM((n,), F32)]
                      + [pltpu.SemaphoreType.DMA] * 10,
        compiler_params=pltpu.CompilerParams(needs_layout_passes=False),
    )
    def gather_k(ta, tb, row, col, x0, x1, x2, outa, outb, outr2,
                 idxr0, idxr1, idxc0, idxc1, bufa0, bufa1, bufb0, bufb1,
                 pka0, pka1, pkb0, pkb1, r2b0, r2b1, xa, xb, xc,
                 sga0, sga1, sgb0, sgb1, swa0, swa1, swb0, swb1, swr0, swr1):
        idxr, idxc = [idxr0, idxr1], [idxc0, idxc1]
        bufa, bufb = [bufa0, bufa1], [bufb0, bufb1]
        pka, pkb = [pka0, pka1], [pkb0, pkb1]
        r2b = [r2b0, r2b1]
        sga, sgb = [sga0, sga1], [sgb0, sgb1]
        swa, swb, swr = [swa0, swa1], [swb0, swb1], [swr0, swr1]

        wid = lax.axis_index("s") * nc + lax.axis_index("c")
        base = wid * epw
        pltpu.sync_copy(x0, xa)
        pltpu.sync_copy(x1, xb)
        pltpu.sync_copy(x2, xc)

        def issue(k, s):
            cb = pl.multiple_of(base + k * chunk, 8)
            pltpu.sync_copy(row.at[pl.ds(cb, chunk)], idxr[s])
            pltpu.sync_copy(col.at[pl.ds(cb, chunk)], idxc[s])
            pltpu.async_copy(ta.at[idxr[s]], bufa[s], sga[s])
            pltpu.async_copy(tb.at[idxc[s]], bufb[s], sgb[s])

        def wait_gathers(s):
            pltpu.make_async_copy(ta.at[idxr[s]], bufa[s], sga[s]).wait()
            pltpu.make_async_copy(tb.at[idxc[s]], bufb[s], sgb[s]).wait()

        def pack(src, dst):
            def pk(kk, carry):
                for par in (0, 1):
                    r = 2 * kk + par
                    for c in range(d // (2 * nl)):
                        va = src[r, pl.ds(c * nl, nl)]
                        vb = src[r, pl.ds(d // 2 + c * nl, nl)]
                        word = _rne16(va) | (_rne16(vb) << 16)
                        dst[kk, pl.ds(par * (d // 2) + c * nl, nl)] = (
                            lax.bitcast_convert_type(word, jnp.int32))
                return carry
            lax.fori_loop(0, chunk // 2, pk, 0)

        def r2comp(s):
            for g in range(groups):
                ir = idxr[s][pl.ds(g * nl, nl)]
                ic = idxc[s][pl.ds(g * nl, nl)]
                dx = plsc.load_gather(xa, [ir]) - plsc.load_gather(xa, [ic])
                dy = plsc.load_gather(xb, [ir]) - plsc.load_gather(xb, [ic])
                dz = plsc.load_gather(xc, [ir]) - plsc.load_gather(xc, [ic])
                r2b[s][pl.ds(g * nl, nl)] = dx * dx + dy * dy + dz * dz

        def flush(k, s):
            cb = pl.multiple_of(base + k * chunk, 8)
            cb2 = pl.multiple_of((base + k * chunk) // 2, 8)
            pltpu.async_copy(pka[s], outa.at[pl.ds(cb2, chunk // 2)], swa[s])
            pltpu.async_copy(pkb[s], outb.at[pl.ds(cb2, chunk // 2)], swb[s])
            pltpu.async_copy(r2b[s], outr2.at[pl.ds(cb, chunk)], swr[s])

        def wait_flush(s):
            z2 = pl.ds(0, chunk // 2)
            pltpu.make_async_copy(pka[s], outa.at[z2], swa[s]).wait()
            pltpu.make_async_copy(pkb[s], outb.at[z2], swb[s]).wait()
            pltpu.make_async_copy(r2b[s], outr2.at[pl.ds(0, chunk)],
                                  swr[s]).wait()

        def step(k, s, first):
            wait_gathers(s)
            pl.when(jnp.logical_not(first))(lambda: wait_flush(s))
            pack(bufa[s], pka[s])
            pack(bufb[s], pkb[s])
            r2comp(s)
            flush(k, s)

        issue(0, 0)

        def body(j2, carry):
            p0 = 2 * j2
            issue(p0 + 1, 1)
            step(p0, 0, j2 < 1)
            issue(p0 + 2, 0)
            step(p0 + 1, 1, j2 < 1)
            return carry

        lax.fori_loop(0, npairs, body, 0)
        step(nchunk - 1, 0, False)
        wait_flush(1)
        wait_flush(0)

    return gather_k


def _make_scatter(n, e, d):
    info = plsc.get_sparse_core_info()
    nc, ns = info.num_cores, info.num_subcores
    nw = nc * ns
    epw = e // nw
    chunk = 80
    nchunk = epw // chunk
    npairs = (nchunk - 1) // 2
    rps = -(-n // (8 * ns)) * 8
    npad = rps * ns
    mesh = plsc.VectorSubcoreMesh(core_axis_name="c", subcore_axis_name="s")

    @functools.partial(
        pl.kernel, mesh=mesh,
        out_type=jax.ShapeDtypeStruct((nc * npad, d), F32),
        scratch_types=[pltpu.VMEM((chunk,), jnp.int32),
                       pltpu.VMEM((chunk,), jnp.int32),
                       pltpu.VMEM((chunk, d), F32),
                       pltpu.VMEM((chunk, d), F32),
                       pltpu.VMEM_SHARED((npad, d), F32)]
                      + [pltpu.SemaphoreType.DMA] * 6,
    )
    def scatter_k(msg, row, zeros, out, idx0, idx1, mb0, mb1, acc,
                  si0, si1, sm0, sm1, sa0, sa1):
        idxv, mbuf = [idx0, idx1], [mb0, mb1]
        si, sm, sa = [si0, si1], [sm0, sm1], [sa0, sa1]
        c = lax.axis_index("c")
        s = lax.axis_index("s")
        wid = s * nc + c
        pltpu.sync_copy(zeros.at[pl.ds(pl.multiple_of(s * rps, 8), rps)],
                        acc.at[pl.ds(pl.multiple_of(s * rps, 8), rps)])
        plsc.subcore_barrier()
        base = wid * epw

        def load(k, sl):
            cb = pl.multiple_of(base + k * chunk, 8)
            pltpu.async_copy(row.at[pl.ds(cb, chunk)], idxv[sl], si[sl])
            pltpu.async_copy(msg.at[pl.ds(cb, chunk)], mbuf[sl], sm[sl])

        def wait_load(sl):
            pltpu.make_async_copy(row.at[pl.ds(0, chunk)], idxv[sl],
                                  si[sl]).wait()
            pltpu.make_async_copy(msg.at[pl.ds(0, chunk)], mbuf[sl],
                                  sm[sl]).wait()

        def add(sl):
            pltpu.async_copy(mbuf[sl], acc.at[idxv[sl]], sa[sl], add=True)

        def wait_add(sl):
            pltpu.make_async_copy(mbuf[sl], acc.at[idxv[sl]], sa[sl]).wait()

        load(0, 0)
        wait_load(0)
        add(0)
        wait_add(0)
        load(1, 1)
        load(2, 0)

        def body(j2, carry):
            pa = 2 * j2 + 1
            wait_load(1)
            add(1)
            wait_load(0)
            add(0)
            wait_add(1)
            pl.when(j2 < npairs - 1)(lambda: load(pa + 2, 1))
            wait_add(0)
            pl.when(j2 < npairs - 1)(lambda: load(pa + 3, 0))
            return carry

        lax.fori_loop(0, npairs, body, 0)
        plsc.subcore_barrier()
        pltpu.sync_copy(acc.at[pl.ds(pl.multiple_of(s * rps, 8), rps)],
                        out.at[pl.ds(pl.multiple_of(c * npad + s * rps, 8),
                                     rps)])

    return scatter_k, npad


def kernel(x, h, edge_index, edge_attr, We1, be1, We2, be2, Wa, ba,
           Wh1, bh1, Wh2, bh2):
    n, d = h.shape
    e = edge_attr.shape[0]
    de = edge_attr.shape[1]

    row = edge_index[0].astype(jnp.int32)
    col = edge_index[1].astype(jnp.int32)
    xf = x.astype(F32)

    w1a = We1[:d]
    w1b = We1[d:2 * d]
    w1r = We1[2 * d:2 * d + 1]
    w1e = We1[2 * d + 1:]

    nb = 2000
    grid_n = n // nb
    full = lambda shape: pl.BlockSpec(shape, lambda i: tuple(0 for _ in shape))
    rowblk = lambda r, c_: pl.BlockSpec((r, c_), lambda i: (i, 0))

    ta, tb = pl.pallas_call(
        _precompute_body,
        grid=(grid_n,),
        in_specs=[rowblk(nb, d), full((d, 128)), full((d, 128)),
                  full((1, 128))],
        out_specs=[rowblk(nb, 128), rowblk(nb, 128)],
        out_shape=[jax.ShapeDtypeStruct((n, 128), F32),
                   jax.ShapeDtypeStruct((n, 128), F32)],
    )(h, w1a, w1b, be1.reshape(1, 128))

    ga, gb, r2 = _make_gather(n, e, 128)(
        ta, tb, row, col, xf[:, 0], xf[:, 1], xf[:, 2])

    eb = 2560
    nblk = e // eb
    hb = eb // 2
    r2s = r2.reshape(nblk, hb, 2).transpose(0, 2, 1).reshape(nblk, 1, eb)
    eas = edge_attr.reshape(nblk, hb, 2, de).transpose(0, 2, 1, 3)
    eas = eas.reshape(e, de)
    rows = row.reshape(nblk, hb, 2).transpose(0, 2, 1).reshape(e)

    msg = pl.pallas_call(
        _edge_body,
        grid=(nblk,),
        in_specs=[rowblk(hb, 128), rowblk(hb, 128),
                  pl.BlockSpec((1, 1, eb), lambda i: (i, 0, 0)),
                  rowblk(eb, de), full((de, 128)), full((1, 128)),
                  full((128, 128)), full((1, 128)), full((1, 128)),
                  full((1, 1))],
        out_specs=rowblk(eb, 128),
        out_shape=jax.ShapeDtypeStruct((e, 128), F32),
    )(ga, gb, r2s, eas, w1e, w1r, We2,
      be2.reshape(1, 128), Wa.reshape(1, 128), ba.reshape(1, 1))

    scatter_k, npad = _make_scatter(n, e, 128)
    partials = scatter_k(msg, rows, jnp.zeros((npad, 128), F32))
    s0 = partials[:n]
    s1 = partials[npad:npad + n]

    out = pl.pallas_call(
        _node_body,
        grid=(grid_n,),
        in_specs=[rowblk(nb, d), rowblk(nb, 128), rowblk(nb, 128),
                  full((128, 128)), full((128, 128)), full((1, 128)),
                  full((128, 128)), full((1, 128))],
        out_specs=rowblk(nb, d),
        out_shape=jax.ShapeDtypeStruct((n, d), F32),
    )(h, s0, s1, Wh1[:d], Wh1[d:], bh1.reshape(1, 128), Wh2,
      bh2.reshape(1, 128))

    return out


# speedup vs baseline: 1.7272x; 1.7272x over previous
import functools

import jax
import jax.numpy as jnp
from jax import lax
from jax.experimental import pallas as pl
from jax.experimental.pallas import tpu as pltpu
from jax.experimental.pallas import tpu_sc as plsc

F32 = jnp.float32


def _precompute_body(h, w1a, w1b, be1, outa, outb):
    hv = h[...]
    outa[...] = jnp.dot(hv, w1a[...], preferred_element_type=F32) + be1[...]
    outb[...] = jnp.dot(hv, w1b[...], preferred_element_type=F32)


def _edge_body(ga, gb, r2, ea, w1e, w1r, w2, b2, wat, ba, out):
    radial = jnp.transpose(jnp.sqrt(r2[...])[0])
    z = (ga[...] + gb[...] + radial * w1r[...]
         + jnp.dot(ea[...], w1e[...], preferred_element_type=F32))
    m = z * jax.nn.sigmoid(z)
    y = jnp.dot(m, w2[...], preferred_element_type=F32) + b2[...]
    m2 = y * jax.nn.sigmoid(y)
    att_logit = jnp.sum(m2 * wat[...], axis=1, keepdims=True) + ba[...]
    out[...] = m2 * jax.nn.sigmoid(att_logit)


def _node_body(h, s0, s1, wh1a, wh1b, bh1, wh2, bh2, out):
    hv = h[...]
    s = s0[...] + s1[...]
    t = (jnp.dot(hv, wh1a[...], preferred_element_type=F32)
         + jnp.dot(s, wh1b[...], preferred_element_type=F32) + bh1[...])
    t = t * jax.nn.sigmoid(t)
    out[...] = hv + jnp.dot(t, wh2[...], preferred_element_type=F32) + bh2[...]


def _make_gather(n, e, d):
    info = plsc.get_sparse_core_info()
    nc, ns, nl = info.num_cores, info.num_subcores, info.num_lanes
    nw = nc * ns
    epw = e // nw
    chunk = 80
    nchunk = epw // chunk
    npairs = (nchunk - 1) // 2
    groups = chunk // nl
    mesh = plsc.VectorSubcoreMesh(core_axis_name="c", subcore_axis_name="s")

    @functools.partial(
        pl.kernel, mesh=mesh,
        out_type=[jax.ShapeDtypeStruct((e, d), F32),
                  jax.ShapeDtypeStruct((e, d), F32),
                  jax.ShapeDtypeStruct((e,), F32)],
        scratch_types=[pltpu.VMEM((chunk,), jnp.int32),
                       pltpu.VMEM((chunk,), jnp.int32),
                       pltpu.VMEM((chunk,), jnp.int32),
                       pltpu.VMEM((chunk,), jnp.int32),
                       pltpu.VMEM((chunk, d), F32),
                       pltpu.VMEM((chunk, d), F32),
                       pltpu.VMEM((chunk, d), F32),
                       pltpu.VMEM((chunk, d), F32),
                       pltpu.VMEM((chunk,), F32),
                       pltpu.VMEM((chunk,), F32),
                       pltpu.VMEM((n,), F32),
                       pltpu.VMEM((n,), F32),
                       pltpu.VMEM((n,), F32)]
                      + [pltpu.SemaphoreType.DMA] * 10,
        compiler_params=pltpu.CompilerParams(needs_layout_passes=False),
    )
    def gather_k(ta, tb, row, col, x0, x1, x2, outa, outb, outr2,
                 idxr0, idxr1, idxc0, idxc1, bufa0, bufa1, bufb0, bufb1,
                 r2b0, r2b1, xa, xb, xc,
                 sga0, sga1, sgb0, sgb1, swa0, swa1, swb0, swb1, swr0, swr1):
        idxr, idxc = [idxr0, idxr1], [idxc0, idxc1]
        bufa, bufb = [bufa0, bufa1], [bufb0, bufb1]
        r2b = [r2b0, r2b1]
        sga, sgb = [sga0, sga1], [sgb0, sgb1]
        swa, swb, swr = [swa0, swa1], [swb0, swb1], [swr0, swr1]

        wid = lax.axis_index("s") * nc + lax.axis_index("c")
        base = wid * epw
        pltpu.sync_copy(x0, xa)
        pltpu.sync_copy(x1, xb)
        pltpu.sync_copy(x2, xc)

        def issue(k, s):
            cb = pl.multiple_of(base + k * chunk, 8)
            pltpu.sync_copy(row.at[pl.ds(cb, chunk)], idxr[s])
            pltpu.sync_copy(col.at[pl.ds(cb, chunk)], idxc[s])
            pltpu.async_copy(ta.at[idxr[s]], bufa[s], sga[s])
            pltpu.async_copy(tb.at[idxc[s]], bufb[s], sgb[s])

        def wait_gathers(s):
            pltpu.make_async_copy(ta.at[idxr[s]], bufa[s], sga[s]).wait()
            pltpu.make_async_copy(tb.at[idxc[s]], bufb[s], sgb[s]).wait()

        def r2comp(s):
            for g in range(groups):
                ir = idxr[s][pl.ds(g * nl, nl)]
                ic = idxc[s][pl.ds(g * nl, nl)]
                dx = plsc.load_gather(xa, [ir]) - plsc.load_gather(xa, [ic])
                dy = plsc.load_gather(xb, [ir]) - plsc.load_gather(xb, [ic])
                dz = plsc.load_gather(xc, [ir]) - plsc.load_gather(xc, [ic])
                r2b[s][pl.ds(g * nl, nl)] = dx * dx + dy * dy + dz * dz

        def flush(k, s):
            cb = pl.multiple_of(base + k * chunk, 8)
            pltpu.async_copy(bufa[s], outa.at[pl.ds(cb, chunk)], swa[s])
            pltpu.async_copy(bufb[s], outb.at[pl.ds(cb, chunk)], swb[s])
            pltpu.async_copy(r2b[s], outr2.at[pl.ds(cb, chunk)], swr[s])

        def wait_flush(s):
            z2 = pl.ds(0, chunk)
            pltpu.make_async_copy(bufa[s], outa.at[z2], swa[s]).wait()
            pltpu.make_async_copy(bufb[s], outb.at[z2], swb[s]).wait()
            pltpu.make_async_copy(r2b[s], outr2.at[z2], swr[s]).wait()

        issue(0, 0)

        def body(j2, carry):
            p0 = 2 * j2
            pl.when(j2 > 0)(lambda: wait_flush(1))
            issue(p0 + 1, 1)
            wait_gathers(0)
            r2comp(0)
            flush(p0, 0)
            wait_flush(0)
            issue(p0 + 2, 0)
            wait_gathers(1)
            r2comp(1)
            flush(p0 + 1, 1)
            return carry

        lax.fori_loop(0, npairs, body, 0)
        wait_flush(1)
        wait_gathers(0)
        r2comp(0)
        flush(nchunk - 1, 0)
        wait_flush(0)

    return gather_k


def _make_scatter(n, e, d):
    info = plsc.get_sparse_core_info()
    nc, ns = info.num_cores, info.num_subcores
    nw = nc * ns
    epw = e // nw
    chunk = 80
    nchunk = epw // chunk
    npairs = (nchunk - 1) // 2
    rps = -(-n // (8 * ns)) * 8
    npad = rps * ns
    mesh = plsc.VectorSubcoreMesh(core_axis_name="c", subcore_axis_name="s")

    @functools.partial(
        pl.kernel, mesh=mesh,
        out_type=jax.ShapeDtypeStruct((nc * npad, d), F32),
        scratch_types=[pltpu.VMEM((chunk,), jnp.int32),
                       pltpu.VMEM((chunk,), jnp.int32),
                       pltpu.VMEM((chunk, d), F32),
                       pltpu.VMEM((chunk, d), F32),
                       pltpu.VMEM_SHARED((npad, d), F32)]
                      + [pltpu.SemaphoreType.DMA] * 4,
    )
    def scatter_k(msg, row, zeros, out, idx0, idx1, mb0, mb1, acc,
                  si0, si1, sm0, sm1):
        idxv, mbuf = [idx0, idx1], [mb0, mb1]
        si, sm = [si0, si1], [sm0, sm1]
        c = lax.axis_index("c")
        s = lax.axis_index("s")
        wid = s * nc + c
        pltpu.sync_copy(zeros.at[pl.ds(pl.multiple_of(s * rps, 8), rps)],
                        acc.at[pl.ds(pl.multiple_of(s * rps, 8), rps)])
        plsc.subcore_barrier()
        base = wid * epw

        def load(k, sl):
            cb = pl.multiple_of(base + k * chunk, 8)
            pltpu.async_copy(row.at[pl.ds(cb, chunk)], idxv[sl], si[sl])
            pltpu.async_copy(msg.at[pl.ds(cb, chunk)], mbuf[sl], sm[sl])

        def wait_load(sl):
            z1 = pl.ds(0, chunk)
            pltpu.make_async_copy(row.at[z1], idxv[sl], si[sl]).wait()
            pltpu.make_async_copy(msg.at[z1], mbuf[sl], sm[sl]).wait()

        def add(sl):
            pltpu.sync_copy(mbuf[sl], acc.at[idxv[sl]], add=True)

        load(0, 0)

        def body(j2, carry):
            p0 = 2 * j2
            load(p0 + 1, 1)
            wait_load(0)
            add(0)
            load(p0 + 2, 0)
            wait_load(1)
            add(1)
            return carry

        lax.fori_loop(0, npairs, body, 0)
        wait_load(0)
        add(0)
        plsc.subcore_barrier()
        pltpu.sync_copy(acc.at[pl.ds(pl.multiple_of(s * rps, 8), rps)],
                        out.at[pl.ds(pl.multiple_of(c * npad + s * rps, 8),
                                     rps)])

    return scatter_k, npad


def kernel(x, h, edge_index, edge_attr, We1, be1, We2, be2, Wa, ba,
           Wh1, bh1, Wh2, bh2):
    n, d = h.shape
    e = edge_attr.shape[0]
    de = edge_attr.shape[1]

    row = edge_index[0].astype(jnp.int32)
    col = edge_index[1].astype(jnp.int32)
    xf = x.astype(F32)

    w1a = We1[:d]
    w1b = We1[d:2 * d]
    w1r = We1[2 * d:2 * d + 1]
    w1e = We1[2 * d + 1:]

    nb = 2000
    grid_n = n // nb
    full = lambda shape: pl.BlockSpec(shape, lambda i: tuple(0 for _ in shape))
    rowblk = lambda r, c_: pl.BlockSpec((r, c_), lambda i: (i, 0))

    ta, tb = pl.pallas_call(
        _precompute_body,
        grid=(grid_n,),
        in_specs=[rowblk(nb, d), full((d, 128)), full((d, 128)),
                  full((1, 128))],
        out_specs=[rowblk(nb, 128), rowblk(nb, 128)],
        out_shape=[jax.ShapeDtypeStruct((n, 128), F32),
                   jax.ShapeDtypeStruct((n, 128), F32)],
    )(h, w1a, w1b, be1.reshape(1, 128))

    ga, gb, r2 = _make_gather(n, e, 128)(
        ta, tb, row, col, xf[:, 0], xf[:, 1], xf[:, 2])

    eb = 2560
    nblk = e // eb
    msg = pl.pallas_call(
        _edge_body,
        grid=(nblk,),
        in_specs=[rowblk(eb, 128), rowblk(eb, 128),
                  pl.BlockSpec((1, 1, eb), lambda i: (i, 0, 0)),
                  rowblk(eb, de), full((de, 128)), full((1, 128)),
                  full((128, 128)), full((1, 128)), full((1, 128)),
                  full((1, 1))],
        out_specs=rowblk(eb, 128),
        out_shape=jax.ShapeDtypeStruct((e, 128), F32),
    )(ga, gb, r2.reshape(nblk, 1, eb), edge_attr, w1e, w1r, We2,
      be2.reshape(1, 128), Wa.reshape(1, 128), ba.reshape(1, 1))

    scatter_k, npad = _make_scatter(n, e, 128)
    partials = scatter_k(msg, row, jnp.zeros((npad, 128), F32))
    s0 = partials[:n]
    s1 = partials[npad:npad + n]

    out = pl.pallas_call(
        _node_body,
        grid=(grid_n,),
        in_specs=[rowblk(nb, d), rowblk(nb, 128), rowblk(nb, 128),
                  full((128, 128)), full((128, 128)), full((1, 128)),
                  full((128, 128)), full((1, 128))],
        out_specs=rowblk(nb, d),
        out_shape=jax.ShapeDtypeStruct((n, d), F32),
    )(h, s0, s1, Wh1[:d], Wh1[d:], bh1.reshape(1, 128), Wh2,
      bh2.reshape(1, 128))

    return out


# speedup vs baseline: 1.8759x; 1.0861x over previous
import functools

import jax
import jax.numpy as jnp
from jax import lax
from jax.experimental import pallas as pl
from jax.experimental.pallas import tpu as pltpu
from jax.experimental.pallas import tpu_sc as plsc

F32 = jnp.float32


def _precompute_body(h, w1a, w1b, be1, outa, outb):
    hv = h[...]
    outa[...] = jnp.dot(hv, w1a[...], preferred_element_type=F32) + be1[...]
    outb[...] = jnp.dot(hv, w1b[...], preferred_element_type=F32)


def _edge_body(ga, gb, r2, ea, w1e, w1r, w2, b2, wat, ba, out):
    radial = jnp.transpose(jnp.sqrt(r2[...])[0])
    z = (ga[...] + gb[...] + radial * w1r[...]
         + jnp.dot(ea[...], w1e[...], preferred_element_type=F32))
    m = z * jax.nn.sigmoid(z)
    y = jnp.dot(m, w2[...], preferred_element_type=F32) + b2[...]
    m2 = y * jax.nn.sigmoid(y)
    att_logit = jnp.sum(m2 * wat[...], axis=1, keepdims=True) + ba[...]
    out[...] = m2 * jax.nn.sigmoid(att_logit)


def _node_body(h, s0, s1, wh1a, wh1b, bh1, wh2, bh2, out):
    hv = h[...]
    s = s0[...] + s1[...]
    t = (jnp.dot(hv, wh1a[...], preferred_element_type=F32)
         + jnp.dot(s, wh1b[...], preferred_element_type=F32) + bh1[...])
    t = t * jax.nn.sigmoid(t)
    out[...] = hv + jnp.dot(t, wh2[...], preferred_element_type=F32) + bh2[...]


def _make_gather(n, e, d):
    info = plsc.get_sparse_core_info()
    nc, ns, nl = info.num_cores, info.num_subcores, info.num_lanes
    nw = nc * ns
    epw = e // nw
    chunk = 80
    nchunk = epw // chunk
    npairs = (nchunk - 1) // 2
    groups = chunk // nl
    mesh = plsc.VectorSubcoreMesh(core_axis_name="c", subcore_axis_name="s")

    @functools.partial(
        pl.kernel, mesh=mesh,
        out_type=[jax.ShapeDtypeStruct((e, d), F32),
                  jax.ShapeDtypeStruct((e, d), F32),
                  jax.ShapeDtypeStruct((e,), F32)],
        scratch_types=[pltpu.VMEM((chunk,), jnp.int32),
                       pltpu.VMEM((chunk,), jnp.int32),
                       pltpu.VMEM((chunk,), jnp.int32),
                       pltpu.VMEM((chunk,), jnp.int32),
                       pltpu.VMEM((chunk, d), F32),
                       pltpu.VMEM((chunk, d), F32),
                       pltpu.VMEM((chunk, d), F32),
                       pltpu.VMEM((chunk, d), F32),
                       pltpu.VMEM((chunk,), F32),
                       pltpu.VMEM((chunk,), F32),
                       pltpu.VMEM((n,), F32),
                       pltpu.VMEM((n,), F32),
                       pltpu.VMEM((n,), F32)]
                      + [pltpu.SemaphoreType.DMA] * 10,
        compiler_params=pltpu.CompilerParams(needs_layout_passes=False),
    )
    def gather_k(ta, tb, row, col, x0, x1, x2, outa, outb, outr2,
                 idxr0, idxr1, idxc0, idxc1, bufa0, bufa1, bufb0, bufb1,
                 r2b0, r2b1, xa, xb, xc,
                 sga0, sga1, sgb0, sgb1, swa0, swa1, swb0, swb1, swr0, swr1):
        idxr, idxc = [idxr0, idxr1], [idxc0, idxc1]
        bufa, bufb = [bufa0, bufa1], [bufb0, bufb1]
        r2b = [r2b0, r2b1]
        sga, sgb = [sga0, sga1], [sgb0, sgb1]
        swa, swb, swr = [swa0, swa1], [swb0, swb1], [swr0, swr1]

        wid = lax.axis_index("s") * nc + lax.axis_index("c")
        base = wid * epw
        pltpu.sync_copy(x0, xa)
        pltpu.sync_copy(x1, xb)
        pltpu.sync_copy(x2, xc)

        def issue(k, s):
            cb = pl.multiple_of(base + k * chunk, 8)
            pltpu.sync_copy(row.at[pl.ds(cb, chunk)], idxr[s])
            pltpu.sync_copy(col.at[pl.ds(cb, chunk)], idxc[s])
            pltpu.async_copy(ta.at[idxr[s]], bufa[s], sga[s])
            pltpu.async_copy(tb.at[idxc[s]], bufb[s], sgb[s])

        def wait_gathers(s):
            pltpu.make_async_copy(ta.at[idxr[s]], bufa[s], sga[s]).wait()
            pltpu.make_async_copy(tb.at[idxc[s]], bufb[s], sgb[s]).wait()

        def r2comp(s):
            for g in range(groups):
                ir = idxr[s][pl.ds(g * nl, nl)]
                ic = idxc[s][pl.ds(g * nl, nl)]
                dx = plsc.load_gather(xa, [ir]) - plsc.load_gather(xa, [ic])
                dy = plsc.load_gather(xb, [ir]) - plsc.load_gather(xb, [ic])
                dz = plsc.load_gather(xc, [ir]) - plsc.load_gather(xc, [ic])
                r2b[s][pl.ds(g * nl, nl)] = dx * dx + dy * dy + dz * dz

        def flush(k, s):
            cb = pl.multiple_of(base + k * chunk, 8)
            pltpu.async_copy(bufa[s], outa.at[pl.ds(cb, chunk)], swa[s])
            pltpu.async_copy(bufb[s], outb.at[pl.ds(cb, chunk)], swb[s])
            pltpu.async_copy(r2b[s], outr2.at[pl.ds(cb, chunk)], swr[s])

        def wait_flush(s):
            z2 = pl.ds(0, chunk)
            pltpu.make_async_copy(bufa[s], outa.at[z2], swa[s]).wait()
            pltpu.make_async_copy(bufb[s], outb.at[z2], swb[s]).wait()
            pltpu.make_async_copy(r2b[s], outr2.at[z2], swr[s]).wait()

        issue(0, 0)

        def body(j2, carry):
            p0 = 2 * j2
            pl.when(j2 > 0)(lambda: wait_flush(1))
            issue(p0 + 1, 1)
            wait_gathers(0)
            r2comp(0)
            flush(p0, 0)
            wait_flush(0)
            pl.when(j2 < npairs - 1)(lambda: issue(p0 + 2, 0))
            wait_gathers(1)
            r2comp(1)
            flush(p0 + 1, 1)
            return carry

        lax.fori_loop(0, npairs, body, 0)
        rest = nchunk - 2 * npairs
        wait_flush(1)
        if rest == 2:
            issue(nchunk - 2, 0)
            issue(nchunk - 1, 1)
            wait_gathers(0)
            r2comp(0)
            flush(nchunk - 2, 0)
            wait_gathers(1)
            r2comp(1)
            flush(nchunk - 1, 1)
            wait_flush(1)
        else:
            issue(nchunk - 1, 0)
            wait_gathers(0)
            r2comp(0)
            flush(nchunk - 1, 0)
        wait_flush(0)

    return gather_k


def _make_scatter(n, e, d):
    info = plsc.get_sparse_core_info()
    nc, ns = info.num_cores, info.num_subcores
    nw = nc * ns
    epw = e // nw
    chunk = 80
    nchunk = epw // chunk
    npairs = (nchunk - 1) // 2
    rps = -(-n // (8 * ns)) * 8
    npad = rps * ns
    mesh = plsc.VectorSubcoreMesh(core_axis_name="c", subcore_axis_name="s")

    @functools.partial(
        pl.kernel, mesh=mesh,
        out_type=jax.ShapeDtypeStruct((nc * npad, d), F32),
        scratch_types=[pltpu.VMEM((chunk,), jnp.int32),
                       pltpu.VMEM((chunk,), jnp.int32),
                       pltpu.VMEM((chunk, d), F32),
                       pltpu.VMEM((chunk, d), F32),
                       pltpu.VMEM_SHARED((npad, d), F32)]
                      + [pltpu.SemaphoreType.DMA] * 4,
    )
    def scatter_k(msg, row, init, out, idx0, idx1, mb0, mb1, acc,
                  si0, si1, sm0, sm1):
        idxv, mbuf = [idx0, idx1], [mb0, mb1]
        si, sm = [si0, si1], [sm0, sm1]
        c = lax.axis_index("c")
        s = lax.axis_index("s")
        wid = s * nc + c
        pltpu.sync_copy(
            init.at[pl.ds(pl.multiple_of(c * npad + s * rps, 8), rps)],
            acc.at[pl.ds(pl.multiple_of(s * rps, 8), rps)])
        plsc.subcore_barrier()
        base = wid * epw

        def load(k, sl):
            cb = pl.multiple_of(base + k * chunk, 8)
            pltpu.async_copy(row.at[pl.ds(cb, chunk)], idxv[sl], si[sl])
            pltpu.async_copy(msg.at[pl.ds(cb, chunk)], mbuf[sl], sm[sl])

        def wait_load(sl):
            z1 = pl.ds(0, chunk)
            pltpu.make_async_copy(row.at[z1], idxv[sl], si[sl]).wait()
            pltpu.make_async_copy(msg.at[z1], mbuf[sl], sm[sl]).wait()

        def add(sl):
            pltpu.sync_copy(mbuf[sl], acc.at[idxv[sl]], add=True)

        load(0, 0)

        def body(j2, carry):
            p0 = 2 * j2
            load(p0 + 1, 1)
            wait_load(0)
            add(0)
            pl.when(j2 < npairs - 1)(lambda: load(p0 + 2, 0))
            wait_load(1)
            add(1)
            return carry

        lax.fori_loop(0, npairs, body, 0)
        rest = nchunk - 2 * npairs
        if rest == 2:
            load(nchunk - 2, 0)
            load(nchunk - 1, 1)
            wait_load(0)
            add(0)
            wait_load(1)
            add(1)
        else:
            load(nchunk - 1, 0)
            wait_load(0)
            add(0)
        plsc.subcore_barrier()
        pltpu.sync_copy(acc.at[pl.ds(pl.multiple_of(s * rps, 8), rps)],
                        out.at[pl.ds(pl.multiple_of(c * npad + s * rps, 8),
                                     rps)])

    return scatter_k, npad


def kernel(x, h, edge_index, edge_attr, We1, be1, We2, be2, Wa, ba,
           Wh1, bh1, Wh2, bh2):
    n, d = h.shape
    e = edge_attr.shape[0]
    de = edge_attr.shape[1]

    row = edge_index[0].astype(jnp.int32)
    col = edge_index[1].astype(jnp.int32)
    xf = x.astype(F32)

    w1a = We1[:d]
    w1b = We1[d:2 * d]
    w1r = We1[2 * d:2 * d + 1]
    w1e = We1[2 * d + 1:]

    nb = 2000
    grid_n = n // nb
    full = lambda shape: pl.BlockSpec(shape, lambda i: tuple(0 for _ in shape))
    rowblk = lambda r, c_: pl.BlockSpec((r, c_), lambda i: (i, 0))

    ta, tb = pl.pallas_call(
        _precompute_body,
        grid=(grid_n,),
        in_specs=[rowblk(nb, d), full((d, 128)), full((d, 128)),
                  full((1, 128))],
        out_specs=[rowblk(nb, 128), rowblk(nb, 128)],
        out_shape=[jax.ShapeDtypeStruct((n, 128), F32),
                   jax.ShapeDtypeStruct((n, 128), F32)],
    )(h, w1a, w1b, be1.reshape(1, 128))

    eb = 2560
    units = e // eb
    nseg = 2
    seg_units = [units // nseg + (1 if i < units % nseg else 0)
                 for i in range(nseg)]
    partials = None
    npad = None
    off = 0
    for su in seg_units:
        sz = su * eb
        rs = row[off:off + sz]
        cs = col[off:off + sz]
        ga, gb, r2 = _make_gather(n, sz, 128)(
            ta, tb, rs, cs, xf[:, 0], xf[:, 1], xf[:, 2])
        msg = pl.pallas_call(
            _edge_body,
            grid=(su,),
            in_specs=[rowblk(eb, 128), rowblk(eb, 128),
                      pl.BlockSpec((1, 1, eb), lambda i: (i, 0, 0)),
                      rowblk(eb, de), full((de, 128)), full((1, 128)),
                      full((128, 128)), full((1, 128)), full((1, 128)),
                      full((1, 1))],
            out_specs=rowblk(eb, 128),
            out_shape=jax.ShapeDtypeStruct((sz, 128), F32),
        )(ga, gb, r2.reshape(su, 1, eb), edge_attr[off:off + sz], w1e, w1r,
          We2, be2.reshape(1, 128), Wa.reshape(1, 128), ba.reshape(1, 1))
        scatter_k, npad = _make_scatter(n, sz, 128)
        init = (jnp.zeros((2 * npad, 128), F32) if partials is None
                else partials)
        partials = scatter_k(msg, rs, init)
        off += sz
    s0 = partials[:n]
    s1 = partials[npad:npad + n]

    out = pl.pallas_call(
        _node_body,
        grid=(grid_n,),
        in_specs=[rowblk(nb, d), rowblk(nb, 128), rowblk(nb, 128),
                  full((128, 128)), full((128, 128)), full((1, 128)),
                  full((128, 128)), full((1, 128))],
        out_specs=rowblk(nb, d),
        out_shape=jax.ShapeDtypeStruct((n, d), F32),
    )(h, s0, s1, Wh1[:d], Wh1[d:], bh1.reshape(1, 128), Wh2,
      bh2.reshape(1, 128))

    return out


# speedup vs baseline: 1.9189x; 1.0229x over previous
import functools

import jax
import jax.numpy as jnp
from jax import lax
from jax.experimental import pallas as pl
from jax.experimental.pallas import tpu as pltpu
from jax.experimental.pallas import tpu_sc as plsc

F32 = jnp.float32


def _precompute_body(h, w1a, w1b, be1, outa, outb):
    hv = h[...]
    outa[...] = jnp.dot(hv, w1a[...], preferred_element_type=F32) + be1[...]
    outb[...] = jnp.dot(hv, w1b[...], preferred_element_type=F32)


def _edge_body(ga, gb, r2, ea, w1e, w1r, w2, b2, wat, ba, out):
    radial = jnp.transpose(jnp.sqrt(r2[...])[0])
    z = (ga[...] + gb[...] + radial * w1r[...]
         + jnp.dot(ea[...], w1e[...], preferred_element_type=F32))
    m = z * jax.nn.sigmoid(z)
    y = jnp.dot(m, w2[...], preferred_element_type=F32) + b2[...]
    m2 = y * jax.nn.sigmoid(y)
    att_logit = jnp.sum(m2 * wat[...], axis=1, keepdims=True) + ba[...]
    out[...] = m2 * jax.nn.sigmoid(att_logit)


def _node_body(h, s0, s1, wh1a, wh1b, bh1, wh2, bh2, out):
    hv = h[...]
    s = s0[...] + s1[...]
    t = (jnp.dot(hv, wh1a[...], preferred_element_type=F32)
         + jnp.dot(s, wh1b[...], preferred_element_type=F32) + bh1[...])
    t = t * jax.nn.sigmoid(t)
    out[...] = hv + jnp.dot(t, wh2[...], preferred_element_type=F32) + bh2[...]


def _make_gather(n, e, d):
    info = plsc.get_sparse_core_info()
    nc, ns, nl = info.num_cores, info.num_subcores, info.num_lanes
    nw = nc * ns
    epw = e // nw
    chunk = 80
    nchunk = epw // chunk
    npairs = (nchunk - 1) // 2
    groups = chunk // nl
    mesh = plsc.VectorSubcoreMesh(core_axis_name="c", subcore_axis_name="s")

    @functools.partial(
        pl.kernel, mesh=mesh,
        out_type=[jax.ShapeDtypeStruct((e, d), F32),
                  jax.ShapeDtypeStruct((e, d), F32),
                  jax.ShapeDtypeStruct((e,), F32)],
        scratch_types=[pltpu.VMEM((chunk,), jnp.int32),
                       pltpu.VMEM((chunk,), jnp.int32),
                       pltpu.VMEM((chunk,), jnp.int32),
                       pltpu.VMEM((chunk,), jnp.int32),
                       pltpu.VMEM((chunk, d), F32),
                       pltpu.VMEM((chunk, d), F32),
                       pltpu.VMEM((chunk, d), F32),
                       pltpu.VMEM((chunk, d), F32),
                       pltpu.VMEM((chunk,), F32),
                       pltpu.VMEM((chunk,), F32),
                       pltpu.VMEM((n,), F32),
                       pltpu.VMEM((n,), F32),
                       pltpu.VMEM((n,), F32)]
                      + [pltpu.SemaphoreType.DMA] * 10,
        compiler_params=pltpu.CompilerParams(needs_layout_passes=False),
    )
    def gather_k(ta, tb, row, col, x0, x1, x2, outa, outb, outr2,
                 idxr0, idxr1, idxc0, idxc1, bufa0, bufa1, bufb0, bufb1,
                 r2b0, r2b1, xa, xb, xc,
                 sga0, sga1, sgb0, sgb1, swa0, swa1, swb0, swb1, swr0, swr1):
        idxr, idxc = [idxr0, idxr1], [idxc0, idxc1]
        bufa, bufb = [bufa0, bufa1], [bufb0, bufb1]
        r2b = [r2b0, r2b1]
        sga, sgb = [sga0, sga1], [sgb0, sgb1]
        swa, swb, swr = [swa0, swa1], [swb0, swb1], [swr0, swr1]

        wid = lax.axis_index("s") * nc + lax.axis_index("c")
        base = wid * epw
        pltpu.sync_copy(x0, xa)
        pltpu.sync_copy(x1, xb)
        pltpu.sync_copy(x2, xc)

        def issue(k, s):
            cb = pl.multiple_of(base + k * chunk, 8)
            pltpu.sync_copy(row.at[pl.ds(cb, chunk)], idxr[s])
            pltpu.sync_copy(col.at[pl.ds(cb, chunk)], idxc[s])
            pltpu.async_copy(ta.at[idxr[s]], bufa[s], sga[s])
            pltpu.async_copy(tb.at[idxc[s]], bufb[s], sgb[s])

        def wait_gathers(s):
            pltpu.make_async_copy(ta.at[idxr[s]], bufa[s], sga[s]).wait()
            pltpu.make_async_copy(tb.at[idxc[s]], bufb[s], sgb[s]).wait()

        def r2comp(s):
            for g in range(groups):
                ir = idxr[s][pl.ds(g * nl, nl)]
                ic = idxc[s][pl.ds(g * nl, nl)]
                dx = plsc.load_gather(xa, [ir]) - plsc.load_gather(xa, [ic])
                dy = plsc.load_gather(xb, [ir]) - plsc.load_gather(xb, [ic])
                dz = plsc.load_gather(xc, [ir]) - plsc.load_gather(xc, [ic])
                r2b[s][pl.ds(g * nl, nl)] = dx * dx + dy * dy + dz * dz

        def flush(k, s):
            cb = pl.multiple_of(base + k * chunk, 8)
            pltpu.async_copy(bufa[s], outa.at[pl.ds(cb, chunk)], swa[s])
            pltpu.async_copy(bufb[s], outb.at[pl.ds(cb, chunk)], swb[s])
            pltpu.async_copy(r2b[s], outr2.at[pl.ds(cb, chunk)], swr[s])

        def wait_flush(s):
            z2 = pl.ds(0, chunk)
            pltpu.make_async_copy(bufa[s], outa.at[z2], swa[s]).wait()
            pltpu.make_async_copy(bufb[s], outb.at[z2], swb[s]).wait()
            pltpu.make_async_copy(r2b[s], outr2.at[z2], swr[s]).wait()

        issue(0, 0)

        def body(j2, carry):
            p0 = 2 * j2
            pl.when(j2 > 0)(lambda: wait_flush(1))
            issue(p0 + 1, 1)
            wait_gathers(0)
            r2comp(0)
            flush(p0, 0)
            wait_flush(0)
            pl.when(j2 < npairs - 1)(lambda: issue(p0 + 2, 0))
            wait_gathers(1)
            r2comp(1)
            flush(p0 + 1, 1)
            return carry

        lax.fori_loop(0, npairs, body, 0)
        rest = nchunk - 2 * npairs
        wait_flush(1)
        if rest == 2:
            issue(nchunk - 2, 0)
            issue(nchunk - 1, 1)
            wait_gathers(0)
            r2comp(0)
            flush(nchunk - 2, 0)
            wait_gathers(1)
            r2comp(1)
            flush(nchunk - 1, 1)
            wait_flush(1)
        else:
            issue(nchunk - 1, 0)
            wait_gathers(0)
            r2comp(0)
            flush(nchunk - 1, 0)
        wait_flush(0)

    return gather_k


def _make_scatter(n, e, d):
    info = plsc.get_sparse_core_info()
    nc, ns = info.num_cores, info.num_subcores
    nw = nc * ns
    epw = e // nw
    chunk = 80
    nchunk = epw // chunk
    npairs = (nchunk - 1) // 2
    rps = -(-n // (8 * ns)) * 8
    npad = rps * ns
    mesh = plsc.VectorSubcoreMesh(core_axis_name="c", subcore_axis_name="s")

    @functools.partial(
        pl.kernel, mesh=mesh,
        out_type=jax.ShapeDtypeStruct((nc * npad, d), F32),
        scratch_types=[pltpu.VMEM((chunk,), jnp.int32),
                       pltpu.VMEM((chunk,), jnp.int32),
                       pltpu.VMEM((chunk, d), F32),
                       pltpu.VMEM((chunk, d), F32),
                       pltpu.VMEM_SHARED((npad, d), F32)]
                      + [pltpu.SemaphoreType.DMA] * 4,
    )
    def scatter_k(msg, row, init, out, idx0, idx1, mb0, mb1, acc,
                  si0, si1, sm0, sm1):
        idxv, mbuf = [idx0, idx1], [mb0, mb1]
        si, sm = [si0, si1], [sm0, sm1]
        c = lax.axis_index("c")
        s = lax.axis_index("s")
        wid = s * nc + c
        pltpu.sync_copy(
            init.at[pl.ds(pl.multiple_of(c * npad + s * rps, 8), rps)],
            acc.at[pl.ds(pl.multiple_of(s * rps, 8), rps)])
        plsc.subcore_barrier()
        base = wid * epw

        def load(k, sl):
            cb = pl.multiple_of(base + k * chunk, 8)
            pltpu.async_copy(row.at[pl.ds(cb, chunk)], idxv[sl], si[sl])
            pltpu.async_copy(msg.at[pl.ds(cb, chunk)], mbuf[sl], sm[sl])

        def wait_load(sl):
            z1 = pl.ds(0, chunk)
            pltpu.make_async_copy(row.at[z1], idxv[sl], si[sl]).wait()
            pltpu.make_async_copy(msg.at[z1], mbuf[sl], sm[sl]).wait()

        def add(sl):
            pltpu.sync_copy(mbuf[sl], acc.at[idxv[sl]], add=True)

        load(0, 0)

        def body(j2, carry):
            p0 = 2 * j2
            load(p0 + 1, 1)
            wait_load(0)
            add(0)
            pl.when(j2 < npairs - 1)(lambda: load(p0 + 2, 0))
            wait_load(1)
            add(1)
            return carry

        lax.fori_loop(0, npairs, body, 0)
        rest = nchunk - 2 * npairs
        if rest == 2:
            load(nchunk - 2, 0)
            load(nchunk - 1, 1)
            wait_load(0)
            add(0)
            wait_load(1)
            add(1)
        else:
            load(nchunk - 1, 0)
            wait_load(0)
            add(0)
        plsc.subcore_barrier()
        pltpu.sync_copy(acc.at[pl.ds(pl.multiple_of(s * rps, 8), rps)],
                        out.at[pl.ds(pl.multiple_of(c * npad + s * rps, 8),
                                     rps)])

    return scatter_k, npad


def kernel(x, h, edge_index, edge_attr, We1, be1, We2, be2, Wa, ba,
           Wh1, bh1, Wh2, bh2):
    n, d = h.shape
    e = edge_attr.shape[0]
    de = edge_attr.shape[1]

    row = edge_index[0].astype(jnp.int32)
    col = edge_index[1].astype(jnp.int32)
    xf = x.astype(F32)

    w1a = We1[:d]
    w1b = We1[d:2 * d]
    w1r = We1[2 * d:2 * d + 1]
    w1e = We1[2 * d + 1:]

    nb = 2000
    grid_n = n // nb
    full = lambda shape: pl.BlockSpec(shape, lambda i: tuple(0 for _ in shape))
    rowblk = lambda r, c_: pl.BlockSpec((r, c_), lambda i: (i, 0))

    ta, tb = pl.pallas_call(
        _precompute_body,
        grid=(grid_n,),
        in_specs=[rowblk(nb, d), full((d, 128)), full((d, 128)),
                  full((1, 128))],
        out_specs=[rowblk(nb, 128), rowblk(nb, 128)],
        out_shape=[jax.ShapeDtypeStruct((n, 128), F32),
                   jax.ShapeDtypeStruct((n, 128), F32)],
    )(h, w1a, w1b, be1.reshape(1, 128))

    eb = 2560
    units = e // eb
    nseg = 3
    seg_units = [units // nseg + (1 if i < units % nseg else 0)
                 for i in range(nseg)]
    partials = None
    npad = None
    off = 0
    for su in seg_units:
        sz = su * eb
        rs = row[off:off + sz]
        cs = col[off:off + sz]
        ga, gb, r2 = _make_gather(n, sz, 128)(
            ta, tb, rs, cs, xf[:, 0], xf[:, 1], xf[:, 2])
        msg = pl.pallas_call(
            _edge_body,
            grid=(su,),
            in_specs=[rowblk(eb, 128), rowblk(eb, 128),
                      pl.BlockSpec((1, 1, eb), lambda i: (i, 0, 0)),
                      rowblk(eb, de), full((de, 128)), full((1, 128)),
                      full((128, 128)), full((1, 128)), full((1, 128)),
                      full((1, 1))],
            out_specs=rowblk(eb, 128),
            out_shape=jax.ShapeDtypeStruct((sz, 128), F32),
        )(ga, gb, r2.reshape(su, 1, eb), edge_attr[off:off + sz], w1e, w1r,
          We2, be2.reshape(1, 128), Wa.reshape(1, 128), ba.reshape(1, 1))
        scatter_k, npad = _make_scatter(n, sz, 128)
        init = (jnp.zeros((2 * npad, 128), F32) if partials is None
                else partials)
        partials = scatter_k(msg, rs, init)
        off += sz
    s0 = partials[:n]
    s1 = partials[npad:npad + n]

    out = pl.pallas_call(
        _node_body,
        grid=(grid_n,),
        in_specs=[rowblk(nb, d), rowblk(nb, 128), rowblk(nb, 128),
                  full((128, 128)), full((128, 128)), full((1, 128)),
                  full((128, 128)), full((1, 128))],
        out_specs=rowblk(nb, d),
        out_shape=jax.ShapeDtypeStruct((n, d), F32),
    )(h, s0, s1, Wh1[:d], Wh1[d:], bh1.reshape(1, 128), Wh2,
      bh2.reshape(1, 128))

    return out


# speedup vs baseline: 1.9196x; 1.0004x over previous
import functools

import jax
import jax.numpy as jnp
from jax import lax
from jax.experimental import pallas as pl
from jax.experimental.pallas import tpu as pltpu
from jax.experimental.pallas import tpu_sc as plsc

F32 = jnp.float32


def _precompute_body(h, w1a, w1b, be1, outa, outb):
    hv = h[...]
    outa[...] = jnp.dot(hv, w1a[...], preferred_element_type=F32) + be1[...]
    outb[...] = jnp.dot(hv, w1b[...], preferred_element_type=F32)


def _edge_body(ga, gb, r2, ea, w1e, w1r, w2, b2, wat, ba, out):
    radial = jnp.transpose(jnp.sqrt(r2[...])[0])
    z = (ga[...] + gb[...] + radial * w1r[...]
         + jnp.dot(ea[...], w1e[...], preferred_element_type=F32))
    m = z * jax.nn.sigmoid(z)
    y = jnp.dot(m, w2[...], preferred_element_type=F32) + b2[...]
    m2 = y * jax.nn.sigmoid(y)
    att_logit = jnp.sum(m2 * wat[...], axis=1, keepdims=True) + ba[...]
    out[...] = m2 * jax.nn.sigmoid(att_logit)


def _node_body(h, s0, s1, wh1a, wh1b, bh1, wh2, bh2, out):
    hv = h[...]
    s = s0[...] + s1[...]
    t = (jnp.dot(hv, wh1a[...], preferred_element_type=F32)
         + jnp.dot(s, wh1b[...], preferred_element_type=F32) + bh1[...])
    t = t * jax.nn.sigmoid(t)
    out[...] = hv + jnp.dot(t, wh2[...], preferred_element_type=F32) + bh2[...]


def _pick_chunk(epw, step):
    for c in range(128 - 128 % step, 0, -step):
        if epw % c == 0:
            return c
    raise ValueError(epw)


def _make_gather(n, e, d):
    info = plsc.get_sparse_core_info()
    nc, ns, nl = info.num_cores, info.num_subcores, info.num_lanes
    nw = nc * ns
    epw = e // nw
    chunk = _pick_chunk(epw, nl)
    nchunk = epw // chunk
    npairs = (nchunk - 1) // 2
    groups = chunk // nl
    mesh = plsc.VectorSubcoreMesh(core_axis_name="c", subcore_axis_name="s")

    @functools.partial(
        pl.kernel, mesh=mesh,
        out_type=[jax.ShapeDtypeStruct((e, d), F32),
                  jax.ShapeDtypeStruct((e, d), F32),
                  jax.ShapeDtypeStruct((e,), F32)],
        scratch_types=[pltpu.VMEM((chunk,), jnp.int32),
                       pltpu.VMEM((chunk,), jnp.int32),
                       pltpu.VMEM((chunk,), jnp.int32),
                       pltpu.VMEM((chunk,), jnp.int32),
                       pltpu.VMEM((chunk, d), F32),
                       pltpu.VMEM((chunk, d), F32),
                       pltpu.VMEM((chunk, d), F32),
                       pltpu.VMEM((chunk, d), F32),
                       pltpu.VMEM((chunk,), F32),
                       pltpu.VMEM((chunk,), F32),
                       pltpu.VMEM((n,), F32),
                       pltpu.VMEM((n,), F32),
                       pltpu.VMEM((n,), F32)]
                      + [pltpu.SemaphoreType.DMA] * 10,
        compiler_params=pltpu.CompilerParams(needs_layout_passes=False),
    )
    def gather_k(ta, tb, row, col, x0, x1, x2, outa, outb, outr2,
                 idxr0, idxr1, idxc0, idxc1, bufa0, bufa1, bufb0, bufb1,
                 r2b0, r2b1, xa, xb, xc,
                 sga0, sga1, sgb0, sgb1, swa0, swa1, swb0, swb1, swr0, swr1):
        idxr, idxc = [idxr0, idxr1], [idxc0, idxc1]
        bufa, bufb = [bufa0, bufa1], [bufb0, bufb1]
        r2b = [r2b0, r2b1]
        sga, sgb = [sga0, sga1], [sgb0, sgb1]
        swa, swb, swr = [swa0, swa1], [swb0, swb1], [swr0, swr1]

        wid = lax.axis_index("s") * nc + lax.axis_index("c")
        base = wid * epw
        pltpu.sync_copy(x0, xa)
        pltpu.sync_copy(x1, xb)
        pltpu.sync_copy(x2, xc)

        def issue(k, s):
            cb = pl.multiple_of(base + k * chunk, 8)
            pltpu.sync_copy(row.at[pl.ds(cb, chunk)], idxr[s])
            pltpu.sync_copy(col.at[pl.ds(cb, chunk)], idxc[s])
            pltpu.async_copy(ta.at[idxr[s]], bufa[s], sga[s])
            pltpu.async_copy(tb.at[idxc[s]], bufb[s], sgb[s])

        def wait_gathers(s):
            pltpu.make_async_copy(ta.at[idxr[s]], bufa[s], sga[s]).wait()
            pltpu.make_async_copy(tb.at[idxc[s]], bufb[s], sgb[s]).wait()

        def r2comp(s):
            for g in range(groups):
                ir = idxr[s][pl.ds(g * nl, nl)]
                ic = idxc[s][pl.ds(g * nl, nl)]
                dx = plsc.load_gather(xa, [ir]) - plsc.load_gather(xa, [ic])
                dy = plsc.load_gather(xb, [ir]) - plsc.load_gather(xb, [ic])
                dz = plsc.load_gather(xc, [ir]) - plsc.load_gather(xc, [ic])
                r2b[s][pl.ds(g * nl, nl)] = dx * dx + dy * dy + dz * dz

        def flush(k, s):
            cb = pl.multiple_of(base + k * chunk, 8)
            pltpu.async_copy(bufa[s], outa.at[pl.ds(cb, chunk)], swa[s])
            pltpu.async_copy(bufb[s], outb.at[pl.ds(cb, chunk)], swb[s])
            pltpu.async_copy(r2b[s], outr2.at[pl.ds(cb, chunk)], swr[s])

        def wait_flush(s):
            z2 = pl.ds(0, chunk)
            pltpu.make_async_copy(bufa[s], outa.at[z2], swa[s]).wait()
            pltpu.make_async_copy(bufb[s], outb.at[z2], swb[s]).wait()
            pltpu.make_async_copy(r2b[s], outr2.at[z2], swr[s]).wait()

        issue(0, 0)

        def body(j2, carry):
            p0 = 2 * j2
            pl.when(j2 > 0)(lambda: wait_flush(1))
            issue(p0 + 1, 1)
            wait_gathers(0)
            r2comp(0)
            flush(p0, 0)
            wait_flush(0)
            pl.when(j2 < npairs - 1)(lambda: issue(p0 + 2, 0))
            wait_gathers(1)
            r2comp(1)
            flush(p0 + 1, 1)
            return carry

        lax.fori_loop(0, npairs, body, 0)
        rest = nchunk - 2 * npairs
        wait_flush(1)
        if rest == 2:
            issue(nchunk - 2, 0)
            issue(nchunk - 1, 1)
            wait_gathers(0)
            r2comp(0)
            flush(nchunk - 2, 0)
            wait_gathers(1)
            r2comp(1)
            flush(nchunk - 1, 1)
            wait_flush(1)
        else:
            issue(nchunk - 1, 0)
            wait_gathers(0)
            r2comp(0)
            flush(nchunk - 1, 0)
        wait_flush(0)

    return gather_k


def _make_scatter(n, e, d):
    info = plsc.get_sparse_core_info()
    nc, ns = info.num_cores, info.num_subcores
    nw = nc * ns
    epw = e // nw
    chunk = _pick_chunk(epw, 8)
    nchunk = epw // chunk
    npairs = (nchunk - 1) // 2
    rps = -(-n // (8 * ns)) * 8
    npad = rps * ns
    mesh = plsc.VectorSubcoreMesh(core_axis_name="c", subcore_axis_name="s")

    @functools.partial(
        pl.kernel, mesh=mesh,
        out_type=jax.ShapeDtypeStruct((nc * npad, d), F32),
        scratch_types=[pltpu.VMEM((chunk,), jnp.int32),
                       pltpu.VMEM((chunk,), jnp.int32),
                       pltpu.VMEM((chunk, d), F32),
                       pltpu.VMEM((chunk, d), F32),
                       pltpu.VMEM_SHARED((npad, d), F32)]
                      + [pltpu.SemaphoreType.DMA] * 4,
    )
    def scatter_k(msg, row, init, out, idx0, idx1, mb0, mb1, acc,
                  si0, si1, sm0, sm1):
        idxv, mbuf = [idx0, idx1], [mb0, mb1]
        si, sm = [si0, si1], [sm0, sm1]
        c = lax.axis_index("c")
        s = lax.axis_index("s")
        wid = s * nc + c
        pltpu.sync_copy(
            init.at[pl.ds(pl.multiple_of(c * npad + s * rps, 8), rps)],
            acc.at[pl.ds(pl.multiple_of(s * rps, 8), rps)])
        plsc.subcore_barrier()
        base = wid * epw

        def load(k, sl):
            cb = pl.multiple_of(base + k * chunk, 8)
            pltpu.async_copy(row.at[pl.ds(cb, chunk)], idxv[sl], si[sl])
            pltpu.async_copy(msg.at[pl.ds(cb, chunk)], mbuf[sl], sm[sl])

        def wait_load(sl):
            z1 = pl.ds(0, chunk)
            pltpu.make_async_copy(row.at[z1], idxv[sl], si[sl]).wait()
            pltpu.make_async_copy(msg.at[z1], mbuf[sl], sm[sl]).wait()

        def add(sl):
            pltpu.sync_copy(mbuf[sl], acc.at[idxv[sl]], add=True)

        load(0, 0)

        def body(j2, carry):
            p0 = 2 * j2
            load(p0 + 1, 1)
            wait_load(0)
            add(0)
            pl.when(j2 < npairs - 1)(lambda: load(p0 + 2, 0))
            wait_load(1)
            add(1)
            return carry

        lax.fori_loop(0, npairs, body, 0)
        rest = nchunk - 2 * npairs
        if rest == 2:
            load(nchunk - 2, 0)
            load(nchunk - 1, 1)
            wait_load(0)
            add(0)
            wait_load(1)
            add(1)
        else:
            load(nchunk - 1, 0)
            wait_load(0)
            add(0)
        plsc.subcore_barrier()
        pltpu.sync_copy(acc.at[pl.ds(pl.multiple_of(s * rps, 8), rps)],
                        out.at[pl.ds(pl.multiple_of(c * npad + s * rps, 8),
                                     rps)])

    return scatter_k, npad


def kernel(x, h, edge_index, edge_attr, We1, be1, We2, be2, Wa, ba,
           Wh1, bh1, Wh2, bh2):
    n, d = h.shape
    e = edge_attr.shape[0]
    de = edge_attr.shape[1]

    row = edge_index[0].astype(jnp.int32)
    col = edge_index[1].astype(jnp.int32)
    xf = x.astype(F32)

    w1a = We1[:d]
    w1b = We1[d:2 * d]
    w1r = We1[2 * d:2 * d + 1]
    w1e = We1[2 * d + 1:]

    nb = 2000
    grid_n = n // nb
    full = lambda shape: pl.BlockSpec(shape, lambda i: tuple(0 for _ in shape))
    rowblk = lambda r, c_: pl.BlockSpec((r, c_), lambda i: (i, 0))

    ta, tb = pl.pallas_call(
        _precompute_body,
        grid=(grid_n,),
        in_specs=[rowblk(nb, d), full((d, 128)), full((d, 128)),
                  full((1, 128))],
        out_specs=[rowblk(nb, 128), rowblk(nb, 128)],
        out_shape=[jax.ShapeDtypeStruct((n, 128), F32),
                   jax.ShapeDtypeStruct((n, 128), F32)],
    )(h, w1a, w1b, be1.reshape(1, 128))

    eb = 2560
    units = e // eb
    nseg = 3
    seg_units = [units // nseg + (1 if i < units % nseg else 0)
                 for i in range(nseg)]
    partials = None
    npad = None
    off = 0
    for su in seg_units:
        sz = su * eb
        rs = row[off:off + sz]
        cs = col[off:off + sz]
        ga, gb, r2 = _make_gather(n, sz, 128)(
            ta, tb, rs, cs, xf[:, 0], xf[:, 1], xf[:, 2])
        msg = pl.pallas_call(
            _edge_body,
            grid=(su,),
            in_specs=[rowblk(eb, 128), rowblk(eb, 128),
                      pl.BlockSpec((1, 1, eb), lambda i: (i, 0, 0)),
                      rowblk(eb, de), full((de, 128)), full((1, 128)),
                      full((128, 128)), full((1, 128)), full((1, 128)),
                      full((1, 1))],
            out_specs=rowblk(eb, 128),
            out_shape=jax.ShapeDtypeStruct((sz, 128), F32),
        )(ga, gb, r2.reshape(su, 1, eb), edge_attr[off:off + sz], w1e, w1r,
          We2, be2.reshape(1, 128), Wa.reshape(1, 128), ba.reshape(1, 1))
        scatter_k, npad = _make_scatter(n, sz, 128)
        init = (jnp.zeros((2 * npad, 128), F32) if partials is None
                else partials)
        partials = scatter_k(msg, rs, init)
        off += sz
    s0 = partials[:n]
    s1 = partials[npad:npad + n]

    out = pl.pallas_call(
        _node_body,
        grid=(grid_n,),
        in_specs=[rowblk(nb, d), rowblk(nb, 128), rowblk(nb, 128),
                  full((128, 128)), full((128, 128)), full((1, 128)),
                  full((128, 128)), full((1, 128))],
        out_specs=rowblk(nb, d),
        out_shape=jax.ShapeDtypeStruct((n, d), F32),
    )(h, s0, s1, Wh1[:d], Wh1[d:], bh1.reshape(1, 128), Wh2,
      bh2.reshape(1, 128))

    return out
